# Initial kernel scaffold; baseline (speedup 1.0000x reference)
import jax
import jax.numpy as jnp
from jax import lax
from jax.experimental import pallas as pl
from jax.experimental.pallas import tpu as pltpu
from jax.experimental.pallas import tpu_sc as plsc

D_IN = 128
D_HID = 128
D_OUT = 64

NP = 10240
NC, NS = 2, 16
NW = NC * NS
CH = 128
K = 84
EPAD = NW * K * CH
ZR = NP // NS
RB = 2048


def _make_prop(d_feat):
    mesh = plsc.VectorSubcoreMesh(core_axis_name="c", subcore_axis_name="s")

    def body(g_hbm, src_hbm, dst_hbm, zeros_hbm, out_hbm,
             src_v, dst_v, rows_v, acc, sem0, sem1, sem2):
        sems = (sem0, sem1, sem2)
        c = lax.axis_index("c")
        s = lax.axis_index("s")
        wid = s * NC + c
        row0 = s * ZR

        pltpu.sync_copy(zeros_hbm, acc.at[pl.ds(row0, ZR)])
        pltpu.sync_copy(src_hbm.at[pl.ds(wid * K, K)], src_v)
        pltpu.sync_copy(dst_hbm.at[pl.ds(wid * K, K)], dst_v)

        def gdesc(j, b):
            return pltpu.make_async_copy(
                g_hbm.at[src_v.at[j]], rows_v.at[b], sems[b])

        for b in range(3):
            gdesc(b, b).start()
        plsc.subcore_barrier()

        def step(jj, carry):
            for b in range(3):
                j = jj * 3 + b
                gdesc(j, b).wait()
                pltpu.sync_copy(rows_v.at[b], acc.at[dst_v.at[j]], add=True)

                @pl.when(jj < K // 3 - 1)
                def _():
                    gdesc(j + 3, b).start()
            return carry

        lax.fori_loop(0, K // 3, step, 0)

        plsc.subcore_barrier()
        pltpu.sync_copy(acc.at[pl.ds(row0, ZR)],
                        out_hbm.at[c, pl.ds(row0, ZR)])

    return pl.kernel(
        body,
        out_type=jax.ShapeDtypeStruct((NC, NP, d_feat), jnp.float32),
        mesh=mesh,
        scratch_types=[
            pltpu.VMEM((K, CH), jnp.int32),
            pltpu.VMEM((K, CH), jnp.int32),
            pltpu.VMEM((3, CH, d_feat), jnp.float32),
            pltpu.VMEM_SHARED((NP, d_feat), jnp.float32),
            pltpu.SemaphoreType.DMA,
            pltpu.SemaphoreType.DMA,
            pltpu.SemaphoreType.DMA,
        ],
    )


_DEG_W = 8


def _make_deg():
    mesh = plsc.VectorSubcoreMesh(core_axis_name="c", subcore_axis_name="s")

    def body(dst_hbm, ones_hbm, zeros_hbm, out_hbm,
             dst_v, ones_v, acc, sem0):
        del sem0
        c = lax.axis_index("c")
        s = lax.axis_index("s")
        wid = s * NC + c
        row0 = s * ZR

        pltpu.sync_copy(zeros_hbm, acc.at[pl.ds(row0, ZR)])
        pltpu.sync_copy(dst_hbm.at[pl.ds(wid * K, K)], dst_v)
        pltpu.sync_copy(ones_hbm, ones_v)
        plsc.subcore_barrier()

        def step(j, carry):
            pltpu.sync_copy(ones_v, acc.at[dst_v.at[j]], add=True)
            return carry

        lax.fori_loop(0, K, step, 0)

        plsc.subcore_barrier()
        pltpu.sync_copy(acc.at[pl.ds(row0, ZR)],
                        out_hbm.at[c, pl.ds(row0, ZR)])

    return pl.kernel(
        body,
        out_type=jax.ShapeDtypeStruct((NC, NP, _DEG_W), jnp.float32),
        mesh=mesh,
        scratch_types=[
            pltpu.VMEM((K, CH), jnp.int32),
            pltpu.VMEM((CH, _DEG_W), jnp.float32),
            pltpu.VMEM_SHARED((NP, _DEG_W), jnp.float32),
            pltpu.SemaphoreType.DMA,
        ],
    )


def _dinv_of(deg_ref):
    d = deg_ref[0, :, 0:1] + deg_ref[1, :, 0:1]
    return jnp.where(d > 0, lax.rsqrt(d), 0.0)


def _tc_pre(x_ref, w_ref, deg_ref, o_ref):
    h = jnp.dot(x_ref[...], w_ref[...], preferred_element_type=jnp.float32)
    o_ref[...] = _dinv_of(deg_ref) * h


def _tc_mid(acc_ref, deg_ref, b1_ref, w_ref, o_ref):
    dinv = _dinv_of(deg_ref)
    s = acc_ref[0] + acc_ref[1]
    h1 = jnp.maximum(dinv * s + b1_ref[...], 0.0)
    o_ref[...] = dinv * jnp.dot(h1, w_ref[...],
                                preferred_element_type=jnp.float32)


def _tc_post(acc_ref, deg_ref, b2_ref, o_ref):
    dinv = _dinv_of(deg_ref)
    o = dinv * (acc_ref[0] + acc_ref[1]) + b2_ref[...]
    m = jnp.max(o, axis=1, keepdims=True)
    lse = m + jnp.log(jnp.sum(jnp.exp(o - m), axis=1, keepdims=True))
    o_ref[...] = o - lse


def kernel(x, edge_index, W1, b1, W2, b2):
    n = x.shape[0]
    nb = NP // RB

    src = edge_index[0].astype(jnp.int32)
    dst = edge_index[1].astype(jnp.int32)
    loop = jnp.arange(n, dtype=jnp.int32)
    npad = EPAD - src.shape[0] - n
    sink = jnp.full((npad,), n, jnp.int32)
    src2d = jnp.concatenate([src, loop, sink]).reshape(NW * K, CH)
    dst2d = jnp.concatenate([dst, loop, sink]).reshape(NW * K, CH)

    x_pad = jnp.zeros((NP, D_IN), jnp.float32).at[:n].set(x)
    zeros_deg = jnp.zeros((ZR, _DEG_W), jnp.float32)
    zeros_hid = jnp.zeros((ZR, D_HID), jnp.float32)
    zeros_out = jnp.zeros((ZR, D_OUT), jnp.float32)
    onehot = (jnp.arange(_DEG_W) == 0).astype(jnp.float32)
    ones_rows = jnp.tile(onehot[None, :], (CH, 1))

    degp = _make_deg()(dst2d, ones_rows, zeros_deg)

    g1 = pl.pallas_call(
        _tc_pre,
        grid=(nb,),
        in_specs=[
            pl.BlockSpec((RB, D_IN), lambda i: (i, 0)),
            pl.BlockSpec((D_IN, D_HID), lambda i: (0, 0)),
            pl.BlockSpec((NC, RB, _DEG_W), lambda i: (0, i, 0)),
        ],
        out_specs=pl.BlockSpec((RB, D_HID), lambda i: (i, 0)),
        out_shape=jax.ShapeDtypeStruct((NP, D_HID), jnp.float32),
    )(x_pad, W1, degp)

    acc1 = _make_prop(D_HID)(g1, src2d, dst2d, zeros_hid)

    g2 = pl.pallas_call(
        _tc_mid,
        grid=(nb,),
        in_specs=[
            pl.BlockSpec((NC, RB, D_HID), lambda i: (0, i, 0)),
            pl.BlockSpec((NC, RB, _DEG_W), lambda i: (0, i, 0)),
            pl.BlockSpec((1, D_HID), lambda i: (0, 0)),
            pl.BlockSpec((D_HID, D_OUT), lambda i: (0, 0)),
        ],
        out_specs=pl.BlockSpec((RB, D_OUT), lambda i: (i, 0)),
        out_shape=jax.ShapeDtypeStruct((NP, D_OUT), jnp.float32),
    )(acc1, degp, b1.reshape(1, D_HID), W2)

    acc2 = _make_prop(D_OUT)(g2, src2d, dst2d, zeros_out)

    out = pl.pallas_call(
        _tc_post,
        grid=(nb,),
        in_specs=[
            pl.BlockSpec((NC, RB, D_OUT), lambda i: (0, i, 0)),
            pl.BlockSpec((NC, RB, _DEG_W), lambda i: (0, i, 0)),
            pl.BlockSpec((1, D_OUT), lambda i: (0, 0)),
        ],
        out_specs=pl.BlockSpec((RB, D_OUT), lambda i: (i, 0)),
        out_shape=jax.ShapeDtypeStruct((NP, D_OUT), jnp.float32),
    )(acc2, degp, b2.reshape(1, D_OUT))

    return out[:n]


# speedup vs baseline: 6.7098x; 6.7098x over previous
import jax
import jax.numpy as jnp
from jax import lax
from jax.experimental import pallas as pl
from jax.experimental.pallas import tpu as pltpu
from jax.experimental.pallas import tpu_sc as plsc

D_IN = 128
D_HID = 128
D_OUT = 64
DP = 128

NP = 10240
NC, NS = 2, 16
NW = NC * NS
CH = 128
GC = 12
NG = 7
K = GC * NG
EPAD = NW * K * CH
ZR = NP // NS
RB = 2048


def _make_prop():
    mesh = plsc.VectorSubcoreMesh(core_axis_name="c", subcore_axis_name="s")

    def body(g_hbm, src_hbm, dst_hbm, zeros_hbm, out_hbm,
             src_v, dst_v, rows_v, acc, gs0, gs1, is0, is1):
        gsems = (gs0, gs1)
        c = lax.axis_index("c")
        s = lax.axis_index("s")
        wid = s * NC + c
        row0 = s * ZR

        pltpu.sync_copy(zeros_hbm, acc.at[pl.ds(row0, ZR)])
        pltpu.sync_copy(src_hbm.at[wid, 0], src_v.at[0])
        pltpu.sync_copy(dst_hbm.at[wid, 0], dst_v.at[0])

        def iload(grp, p):
            return (pltpu.make_async_copy(src_hbm.at[wid, grp],
                                          src_v.at[p], is0),
                    pltpu.make_async_copy(dst_hbm.at[wid, grp],
                                          dst_v.at[p], is1))

        def gdesc(p, j, b):
            return pltpu.make_async_copy(
                g_hbm.at[src_v.at[p, j]], rows_v.at[b], gsems[b])

        for d in iload(1, 1):
            d.start()
        gdesc(0, 0, 0).start()
        gdesc(0, 1, 1).start()
        plsc.subcore_barrier()

        def grp_body(g, carry):
            p = lax.rem(g, 2)
            pn = lax.rem(g + 1, 2)
            for j in range(GC - 2):
                b = j % 2
                gdesc(p, j, b).wait()
                pltpu.sync_copy(rows_v.at[b], acc.at[dst_v.at[p, j]],
                                add=True)
                gdesc(p, j + 2, b).start()

            @pl.when(g < NG - 1)
            def _():
                for d in iload(g + 1, pn):
                    d.wait()

            for j in (GC - 2, GC - 1):
                b = j % 2
                gdesc(p, j, b).wait()
                pltpu.sync_copy(rows_v.at[b], acc.at[dst_v.at[p, j]],
                                add=True)

                @pl.when(g < NG - 1)
                def _():
                    gdesc(pn, j - (GC - 2), b).start()

            @pl.when(g < NG - 2)
            def _():
                for d in iload(g + 2, p):
                    d.start()

            return carry

        lax.fori_loop(0, NG, grp_body, 0)

        plsc.subcore_barrier()
        pltpu.sync_copy(acc.at[pl.ds(row0, ZR)],
                        out_hbm.at[c, pl.ds(row0, ZR)])

    return pl.kernel(
        body,
        out_type=jax.ShapeDtypeStruct((NC, NP, DP), jnp.float32),
        mesh=mesh,
        scratch_types=[
            pltpu.VMEM((2, GC, CH), jnp.int32),
            pltpu.VMEM((2, GC, CH), jnp.int32),
            pltpu.VMEM((2, CH, DP), jnp.float32),
            pltpu.VMEM_SHARED((NP, DP), jnp.float32),
            pltpu.SemaphoreType.DMA,
            pltpu.SemaphoreType.DMA,
            pltpu.SemaphoreType.DMA,
            pltpu.SemaphoreType.DMA,
        ],
    )


def _make_deg():
    mesh = plsc.VectorSubcoreMesh(core_axis_name="c", subcore_axis_name="s")

    def body(dst_hbm, ones_hbm, zeros_hbm, out_hbm, dst_v, ones_v, acc):
        c = lax.axis_index("c")
        s = lax.axis_index("s")
        wid = s * NC + c
        row0 = s * ZR

        pltpu.sync_copy(zeros_hbm, acc.at[pl.ds(row0, ZR)])
        pltpu.sync_copy(dst_hbm.at[wid], dst_v)
        pltpu.sync_copy(ones_hbm, ones_v)
        plsc.subcore_barrier()

        def step(j, carry):
            pltpu.sync_copy(ones_v, acc.at[dst_v.at[j]], add=True)
            return carry

        lax.fori_loop(0, K, step, 0)

        plsc.subcore_barrier()
        pltpu.sync_copy(acc.at[pl.ds(row0, ZR)],
                        out_hbm.at[c, pl.ds(row0, ZR)])

    return pl.kernel(
        body,
        out_type=jax.ShapeDtypeStruct((NC, NP), jnp.float32),
        mesh=mesh,
        scratch_types=[
            pltpu.VMEM((K, CH), jnp.int32),
            pltpu.VMEM((CH,), jnp.float32),
            pltpu.VMEM_SHARED((NP,), jnp.float32),
        ],
    )


def _dinv_of(deg_ref):
    d = deg_ref[0] + deg_ref[1]
    return jnp.where(d > 0, lax.rsqrt(d), 0.0)


def _tc_pre(x_ref, w_ref, deg_ref, o_ref):
    h = jnp.dot(x_ref[...], w_ref[...], preferred_element_type=jnp.float32)
    o_ref[...] = _dinv_of(deg_ref) * h


def _tc_mid(acc_ref, deg_ref, b1_ref, w_ref, o_ref):
    dinv = _dinv_of(deg_ref)
    s = acc_ref[0] + acc_ref[1]
    h1 = jnp.maximum(dinv * s + b1_ref[...], 0.0)
    g2 = dinv * jnp.dot(h1, w_ref[...], preferred_element_type=jnp.float32)
    o_ref[...] = jnp.concatenate(
        [g2, jnp.zeros((RB, DP - D_OUT), jnp.float32)], axis=1)


def _tc_post(acc_ref, deg_ref, b2_ref, o_ref):
    dinv = _dinv_of(deg_ref)
    o = dinv * (acc_ref[0, :, :D_OUT] + acc_ref[1, :, :D_OUT]) + b2_ref[...]
    m = jnp.max(o, axis=1, keepdims=True)
    lse = m + jnp.log(jnp.sum(jnp.exp(o - m), axis=1, keepdims=True))
    o_ref[...] = o - lse


def kernel(x, edge_index, W1, b1, W2, b2):
    n = x.shape[0]
    nb = NP // RB

    src = edge_index[0].astype(jnp.int32)
    dst = edge_index[1].astype(jnp.int32)
    loop = jnp.arange(n, dtype=jnp.int32)
    npad = EPAD - src.shape[0] - n
    sink = jnp.full((npad,), n, jnp.int32)
    src4d = jnp.concatenate([src, loop, sink]).reshape(NW, NG, GC, CH)
    dst4d = jnp.concatenate([dst, loop, sink]).reshape(NW, NG, GC, CH)
    dst32 = dst4d.reshape(NW, K, CH)

    x_pad = jnp.zeros((NP, D_IN), jnp.float32).at[:n].set(x)
    zeros_deg = jnp.zeros((ZR,), jnp.float32)
    zeros_feat = jnp.zeros((ZR, DP), jnp.float32)
    ones_elems = jnp.ones((CH,), jnp.float32)

    degp = _make_deg()(dst32, ones_elems, zeros_deg).reshape(NC, NP, 1)

    g1 = pl.pallas_call(
        _tc_pre,
        grid=(nb,),
        in_specs=[
            pl.BlockSpec((RB, D_IN), lambda i: (i, 0)),
            pl.BlockSpec((D_IN, D_HID), lambda i: (0, 0)),
            pl.BlockSpec((NC, RB, 1), lambda i: (0, i, 0)),
        ],
        out_specs=pl.BlockSpec((RB, D_HID), lambda i: (i, 0)),
        out_shape=jax.ShapeDtypeStruct((NP, D_HID), jnp.float32),
    )(x_pad, W1, degp)

    prop = _make_prop()
    acc1 = prop(g1, src4d, dst4d, zeros_feat)

    g2 = pl.pallas_call(
        _tc_mid,
        grid=(nb,),
        in_specs=[
            pl.BlockSpec((NC, RB, D_HID), lambda i: (0, i, 0)),
            pl.BlockSpec((NC, RB, 1), lambda i: (0, i, 0)),
            pl.BlockSpec((1, D_HID), lambda i: (0, 0)),
            pl.BlockSpec((D_HID, D_OUT), lambda i: (0, 0)),
        ],
        out_specs=pl.BlockSpec((RB, DP), lambda i: (i, 0)),
        out_shape=jax.ShapeDtypeStruct((NP, DP), jnp.float32),
    )(acc1, degp, b1.reshape(1, D_HID), W2)

    acc2 = prop(g2, src4d, dst4d, zeros_feat)

    out = pl.pallas_call(
        _tc_post,
        grid=(nb,),
        in_specs=[
            pl.BlockSpec((NC, RB, DP), lambda i: (0, i, 0)),
            pl.BlockSpec((NC, RB, 1), lambda i: (0, i, 0)),
            pl.BlockSpec((1, D_OUT), lambda i: (0, 0)),
        ],
        out_specs=pl.BlockSpec((RB, D_OUT), lambda i: (i, 0)),
        out_shape=jax.ShapeDtypeStruct((NP, D_OUT), jnp.float32),
    )(acc2, degp, b2.reshape(1, D_OUT))

    return out[:n]
